# baseline (speedup 1.0000x reference)
import functools

import jax
import jax.numpy as jnp
from jax import lax
from jax.experimental import pallas as pl
from jax.experimental.pallas import tpu as pltpu
from jax.experimental.pallas import tpu_sc as plsc

_LANES = 16
_NC = 2
_NS = 16
_CH = 128


@functools.lru_cache(maxsize=None)
def _build(B, D):
    NW = _NC * _NS
    bpw = B // NW
    nch = bpw // _CH
    mesh = plsc.VectorSubcoreMesh(core_axis_name="c", subcore_axis_name="s")

    @functools.partial(
        pl.kernel,
        mesh=mesh,
        compiler_params=pltpu.CompilerParams(
            needs_layout_passes=False, use_tc_tiling_on_sc=False),
        out_type=jax.ShapeDtypeStruct((B,), jnp.float32),
        scratch_types=[
            pltpu.VMEM((nch, _CH), jnp.int32),
            pltpu.VMEM((nch, _CH), jnp.int32),
            pltpu.VMEM((bpw, D), jnp.float32),
            pltpu.VMEM((bpw, D), jnp.float32),
            pltpu.VMEM((bpw * _LANES,), jnp.float32),
            pltpu.VMEM((bpw,), jnp.float32),
            pltpu.SemaphoreType.DMA,
            pltpu.SemaphoreType.DMA,
        ],
    )
    def sc_kernel(user_hbm, item_hbm, uemb_hbm, iemb_hbm, out_hbm,
                  idx_u, idx_i, rows_u, rows_i, part, outv, sem_u, sem_i):
        wid = lax.axis_index("s") * _NC + lax.axis_index("c")
        base = wid * bpw

        pltpu.sync_copy(user_hbm.at[wid], idx_u)
        pltpu.sync_copy(item_hbm.at[wid], idx_i)

        copies = []
        for j in range(nch):
            copies.append(pltpu.async_copy(
                uemb_hbm.at[idx_u.at[j]], rows_u.at[pl.ds(j * _CH, _CH)], sem_u))
            copies.append(pltpu.async_copy(
                iemb_hbm.at[idx_i.at[j]], rows_i.at[pl.ds(j * _CH, _CH)], sem_i))
        for cp in copies:
            cp.wait()

        nt = D // _LANES

        def row_body(r, carry):
            acc = rows_u[r, pl.ds(0, _LANES)] * rows_i[r, pl.ds(0, _LANES)]
            for t in range(1, nt):
                acc = acc + (rows_u[r, pl.ds(t * _LANES, _LANES)]
                             * rows_i[r, pl.ds(t * _LANES, _LANES)])
            part[pl.ds(r * _LANES, _LANES)] = acc
            return carry

        lax.fori_loop(0, bpw, row_body, 0)

        lane = lax.iota(jnp.int32, 16) * _LANES

        def grp_body(g, carry):
            gbase = g * (_LANES * _LANES)
            acc = plsc.load_gather(part, [gbase + lane])
            for l in range(1, _LANES):
                acc = acc + plsc.load_gather(part, [gbase + lane + l])
            outv[pl.ds(g * _LANES, _LANES)] = acc
            return carry

        lax.fori_loop(0, bpw // _LANES, grp_body, 0)

        pltpu.sync_copy(outv, out_hbm.at[pl.ds(base, bpw)])

    return sc_kernel


def kernel(user, item, user_emb, item_emb):
    B = user.shape[0]
    D = user_emb.shape[1]
    NW = _NC * _NS
    nch = (B // NW) // _CH
    user_r = user.astype(jnp.int32).reshape(NW, nch, _CH)
    item_r = item.astype(jnp.int32).reshape(NW, nch, _CH)
    return _build(B, D)(user_r, item_r, user_emb, item_emb)
